# baseline (speedup 1.0000x reference)
import functools
import math

import jax
import jax.numpy as jnp
from jax import lax
from jax.experimental import pallas as pl
from jax.experimental.pallas import tpu as pltpu
from jax.experimental.pallas import tpu_sc as plsc

_VOCAB = 1000000
_EMB = 64
_SCALE = math.sqrt(_EMB)

_NC = 2
_NS = 16
_NW = _NC * _NS

_B = 1024 * 200 * 4
_BPW = _B // _NW
_CHUNK = 128
_NCHUNK = _BPW // _CHUNK

_mesh = plsc.VectorSubcoreMesh(core_axis_name="c", subcore_axis_name="s")


@functools.partial(
    pl.kernel,
    mesh=_mesh,
    out_type=jax.ShapeDtypeStruct((_B, _EMB), jnp.float32),
    scratch_types=[
        pltpu.VMEM((_NCHUNK, _CHUNK), jnp.int32),
        pltpu.VMEM((_CHUNK, _EMB), jnp.float32),
        pltpu.SemaphoreType.DMA,
    ],
    compiler_params=pltpu.CompilerParams(use_tc_tiling_on_sc=False),
)
def _emb_lookup(idx_hbm, table_hbm, out_hbm, idx_v, rows_v, sem):
    wid = lax.axis_index("s") * _NC + lax.axis_index("c")
    base = wid * _BPW
    pltpu.sync_copy(idx_hbm.at[wid], idx_v)

    def chunk_body(j, carry):
        pltpu.async_copy(table_hbm.at[idx_v.at[j]], rows_v, sem).wait()

        def row_body(i, c):
            for t in range(_EMB // 16):
                sl = pl.ds(t * 16, 16)
                rows_v[i, sl] = rows_v[i, sl] * _SCALE
            return c

        lax.fori_loop(0, _CHUNK, row_body, 0, unroll=4)

        pltpu.sync_copy(rows_v, out_hbm.at[pl.ds(base + j * _CHUNK, _CHUNK)])
        return carry

    lax.fori_loop(0, _NCHUNK, chunk_body, 0)


def kernel(content, table):
    bs, l, sub = content.shape
    idx = content.astype(jnp.int32).reshape(_NW, _NCHUNK, _CHUNK)
    out = _emb_lookup(idx, table)
    return out.reshape(bs, l, sub * _EMB)


# speedup vs baseline: 1.0681x; 1.0681x over previous
import functools
import math

import jax
import jax.numpy as jnp
from jax import lax
from jax.experimental import pallas as pl
from jax.experimental.pallas import tpu as pltpu
from jax.experimental.pallas import tpu_sc as plsc

_VOCAB = 1000000
_EMB = 64
_SCALE = math.sqrt(_EMB)

_NC = 2
_NS = 16
_NW = _NC * _NS

_B = 1024 * 200 * 4
_BPW = _B // _NW
_CHUNK = 128
_NCHUNK = _BPW // _CHUNK
_NBUF = 4
_NGRP = _NCHUNK // _NBUF

_mesh = plsc.VectorSubcoreMesh(core_axis_name="c", subcore_axis_name="s")


@functools.partial(
    pl.kernel,
    mesh=_mesh,
    out_type=jax.ShapeDtypeStruct((_B, _EMB), jnp.float32),
    scratch_types=[
        pltpu.VMEM((_NCHUNK, _CHUNK), jnp.int32),
        pltpu.VMEM((_NBUF, _CHUNK, _EMB), jnp.float32),
        pltpu.VMEM((_NBUF, _CHUNK, _EMB), jnp.float32),
        pltpu.SemaphoreType.DMA((_NBUF,)),
        pltpu.SemaphoreType.DMA((_NBUF,)),
    ],
    compiler_params=pltpu.CompilerParams(use_tc_tiling_on_sc=False),
)
def _emb_lookup(idx_hbm, table_hbm, out_hbm, idx_v, gbuf, sbuf, gsem, ssem):
    wid = lax.axis_index("s") * _NC + lax.axis_index("c")
    base = wid * _BPW
    pltpu.sync_copy(idx_hbm.at[wid], idx_v)

    def gather_start(j, b):
        pltpu.async_copy(table_hbm.at[idx_v.at[j]], gbuf.at[b], gsem.at[b])

    def gather_wait(b):
        pltpu.make_async_copy(
            table_hbm.at[idx_v.at[0]], gbuf.at[b], gsem.at[b]).wait()

    def scatter_start(j, b):
        pltpu.async_copy(
            sbuf.at[b], out_hbm.at[pl.ds(base + j * _CHUNK, _CHUNK)],
            ssem.at[b])

    def scatter_wait(b):
        pltpu.make_async_copy(
            sbuf.at[b], out_hbm.at[pl.ds(base, _CHUNK)], ssem.at[b]).wait()

    def scale(b):
        def row_body(i, c):
            for t in range(_EMB // 16):
                sl = pl.ds(t * 16, 16)
                sbuf[b, i, sl] = gbuf[b, i, sl] * _SCALE
            return c
        lax.fori_loop(0, _CHUNK, row_body, 0, unroll=4)

    for b in range(_NBUF):
        gather_start(b, b)

    def group(g, carry):
        for b in range(_NBUF):
            j = g * _NBUF + b
            gather_wait(b)
            @pl.when(g > 0)
            def _():
                scatter_wait(b)
            scale(b)
            @pl.when(j + _NBUF < _NCHUNK)
            def _():
                gather_start(j + _NBUF, b)
            scatter_start(j, b)
        return carry

    lax.fori_loop(0, _NGRP, group, 0)

    for b in range(_NBUF):
        scatter_wait(b)


def kernel(content, table):
    bs, l, sub = content.shape
    idx = content.astype(jnp.int32).reshape(_NW, _NCHUNK, _CHUNK)
    out = _emb_lookup(idx, table)
    return out.reshape(bs, l, sub * _EMB)
